# baseline (speedup 1.0000x reference)
import functools

import jax
import jax.numpy as jnp
import numpy as np
from jax import lax
from jax.experimental import pallas as pl
from jax.experimental.pallas import tpu as pltpu
from jax.experimental.pallas import tpu_sc as plsc

NUM_LEVELS = 12
HASHMAP_SIZE = 2 ** 15
_b = np.exp((np.log(512) - np.log(16)) / (NUM_LEVELS - 1))
RESOLUTIONS = [int(16 * _b ** i) for i in range(NUM_LEVELS)]
P1 = 2654435761 & 0x7FFF
P2 = 805459861 & 0x7FFF

N = 524288
NC, NS, L = 2, 16, 16
NW = NC * NS
PTS_PER_W = N // NW
CP = 1024
NCHUNK = PTS_PER_W // CP
GROUPS = CP // L
ROWS = CP * NUM_LEVELS
PB = 8
BP = CP // PB
BG = GROUPS // PB
TBL = NUM_LEVELS * HASHMAP_SIZE
TSLC = TBL // NS


def _embed(xt_flat, emb_words):
    mesh = plsc.VectorSubcoreMesh(
        core_axis_name="c", subcore_axis_name="s",
        num_cores=NC, num_subcores=NS)

    @functools.partial(
        pl.kernel,
        out_type=jax.ShapeDtypeStruct((NUM_LEVELS * 2, N), jnp.float32),
        mesh=mesh,
        compiler_params=pltpu.CompilerParams(
            needs_layout_passes=False, use_tc_tiling_on_sc=False),
        scratch_types=[
            pltpu.VMEM_SHARED((TBL,), jnp.int32),
            pltpu.VMEM((2 * 3 * CP,), jnp.float32),
            pltpu.VMEM((ROWS,), jnp.int32),
            pltpu.VMEM((ROWS,), jnp.int32),
            pltpu.VMEM((NUM_LEVELS * 2, CP), jnp.float32),
            pltpu.SemaphoreType.DMA,
            pltpu.SemaphoreType.DMA,
            pltpu.SemaphoreType.DMA,
            pltpu.SemaphoreType.DMA,
        ],
    )
    def run(x_hbm, emb_hbm, out_hbm,
            tables, x_buf, idx_buf, word_buf, out_buf,
            gsem0, gsem1, xsem, osem):
        cid = lax.axis_index("c")
        sid = lax.axis_index("s")
        wid = sid * NC + cid

        base = wid * PTS_PER_W

        def fire_x(ci, half):
            p0 = base + ci * CP
            for c in range(3):
                pltpu.async_copy(x_hbm.at[pl.ds(c * N + p0, CP)],
                                 x_buf.at[pl.ds(half * (3 * CP) + c * CP,
                                                CP)], xsem)

        fire_x(0, 0)
        pltpu.sync_copy(emb_hbm.at[pl.ds(sid * TSLC, TSLC)],
                        tables.at[pl.ds(sid * TSLC, TSLC)])
        plsc.subcore_barrier()

        @pl.loop(0, NCHUNK)
        def _chunk(ci):
            p0 = base + ci * CP
            half = ci & 1
            xoff = half * (3 * CP)
            pltpu.make_async_copy(
                x_hbm.at[pl.ds(0, 3 * CP)],
                x_buf.at[pl.ds(xoff, 3 * CP)], xsem).wait()

            @pl.when(ci < NCHUNK - 1)
            def _prefetch():
                fire_x(ci + 1, 1 - half)

            def hash_block(pb):
                @pl.loop(pb * BG, (pb + 1) * BG)
                def _group(g):
                    o = g * L
                    xa = x_buf[pl.ds(xoff + o, L)]
                    xb = x_buf[pl.ds(xoff + CP + o, L)]
                    xc = x_buf[pl.ds(xoff + 2 * CP + o, L)]
                    for lvl in range(NUM_LEVELS):
                        res = float(RESOLUTIONS[lvl])
                        ia = (xa * res).astype(jnp.int32)
                        ib = (xb * res).astype(jnp.int32)
                        ic = (xc * res).astype(jnp.int32)
                        h = (ia ^ (ib * P1) ^ (ic * P2)) & (HASHMAP_SIZE - 1)
                        idx_buf[pl.ds(lvl * CP + o, L)] = h | (lvl << 15)

            def fire_block(pb, sem):
                return [pltpu.async_copy(
                    tables.at[idx_buf.at[pl.ds(lvl * CP + pb * BP, BP)]],
                    word_buf.at[pl.ds(lvl * CP + pb * BP, BP)], sem)
                    for lvl in range(NUM_LEVELS)]

            def unpack_block(pb):
                @pl.loop(pb * BG, (pb + 1) * BG, unroll=2)
                def _unpack(g):
                    o = g * L
                    for lvl in range(NUM_LEVELS):
                        w = word_buf[pl.ds(lvl * CP + o, L)]
                        out_buf[2 * lvl, pl.ds(o, L)] = (
                            plsc.bitcast(w << 16, jnp.float32))
                        out_buf[2 * lvl + 1, pl.ds(o, L)] = (
                            plsc.bitcast(w & jnp.int32(-65536), jnp.float32))

            @pl.when(ci > 0)
            def _drain_out():
                pltpu.make_async_copy(
                    out_buf, out_hbm.at[:, pl.ds(0, CP)], osem).wait()

            def fire_out(pb):
                pltpu.async_copy(
                    out_buf.at[:, pl.ds(pb * BP, BP)],
                    out_hbm.at[:, pl.ds(p0 + pb * BP, BP)], osem)

            sems = (gsem0, gsem1)
            inflight = [None, None]
            hash_block(0)
            inflight[0] = fire_block(0, sems[0])
            for pb in range(1, PB):
                hash_block(pb)
                inflight[pb % 2] = fire_block(pb, sems[pb % 2])
                for cpy in inflight[(pb - 1) % 2]:
                    cpy.wait()
                unpack_block(pb - 1)
                fire_out(pb - 1)
            for cpy in inflight[(PB - 1) % 2]:
                cpy.wait()
            unpack_block(PB - 1)
            fire_out(PB - 1)

        pltpu.make_async_copy(
            out_buf, out_hbm.at[:, pl.ds(0, CP)], osem).wait()

    return run(xt_flat, emb_words)


def kernel(x, embeddings):
    xt_flat = x.T.reshape(3 * N)
    emb_words = lax.bitcast_convert_type(
        embeddings.astype(jnp.bfloat16).reshape(TBL, 2), jnp.int32)
    out = _embed(xt_flat, emb_words)
    return out.T


# speedup vs baseline: 1.1563x; 1.1563x over previous
import functools

import jax
import jax.numpy as jnp
import numpy as np
from jax import lax
from jax.experimental import pallas as pl
from jax.experimental.pallas import tpu as pltpu
from jax.experimental.pallas import tpu_sc as plsc

NUM_LEVELS = 12
HASHMAP_SIZE = 2 ** 15
_b = np.exp((np.log(512) - np.log(16)) / (NUM_LEVELS - 1))
RESOLUTIONS = [int(16 * _b ** i) for i in range(NUM_LEVELS)]
P1 = 2654435761 & 0x7FFF
P2 = 805459861 & 0x7FFF

N = 524288
NC, NS, L = 2, 16, 16
NW = NC * NS
PTS_PER_W = N // NW
CP = 1024
NCHUNK = PTS_PER_W // CP
GROUPS = CP // L
ROWS = CP * NUM_LEVELS
PB = 4
BP = CP // PB
BG = GROUPS // PB
TBL = NUM_LEVELS * HASHMAP_SIZE
TSLC = TBL // NS


def _embed(xt_flat, emb_words):
    mesh = plsc.VectorSubcoreMesh(
        core_axis_name="c", subcore_axis_name="s",
        num_cores=NC, num_subcores=NS)

    @functools.partial(
        pl.kernel,
        out_type=jax.ShapeDtypeStruct((NUM_LEVELS * 2, N), jnp.float32),
        mesh=mesh,
        compiler_params=pltpu.CompilerParams(
            needs_layout_passes=False, use_tc_tiling_on_sc=False),
        scratch_types=[
            pltpu.VMEM_SHARED((TBL,), jnp.int32),
            pltpu.VMEM((2 * 3 * CP,), jnp.float32),
            pltpu.VMEM((ROWS,), jnp.int32),
            pltpu.VMEM((ROWS,), jnp.int32),
            pltpu.VMEM((NUM_LEVELS * 2, CP), jnp.float32),
            pltpu.SemaphoreType.DMA,
            pltpu.SemaphoreType.DMA,
            pltpu.SemaphoreType.DMA,
            pltpu.SemaphoreType.DMA,
        ],
    )
    def run(x_hbm, emb_hbm, out_hbm,
            tables, x_buf, idx_buf, word_buf, out_buf,
            gsem0, gsem1, xsem, osem):
        cid = lax.axis_index("c")
        sid = lax.axis_index("s")
        wid = sid * NC + cid

        base = wid * PTS_PER_W

        def fire_x(ci, half):
            p0 = base + ci * CP
            for c in range(3):
                pltpu.async_copy(x_hbm.at[pl.ds(c * N + p0, CP)],
                                 x_buf.at[pl.ds(half * (3 * CP) + c * CP,
                                                CP)], xsem)

        fire_x(0, 0)
        pltpu.sync_copy(emb_hbm.at[pl.ds(sid * TSLC, TSLC)],
                        tables.at[pl.ds(sid * TSLC, TSLC)])
        plsc.subcore_barrier()

        @pl.loop(0, NCHUNK)
        def _chunk(ci):
            p0 = base + ci * CP
            half = ci & 1
            xoff = half * (3 * CP)
            pltpu.make_async_copy(
                x_hbm.at[pl.ds(0, 3 * CP)],
                x_buf.at[pl.ds(xoff, 3 * CP)], xsem).wait()

            @pl.when(ci < NCHUNK - 1)
            def _prefetch():
                fire_x(ci + 1, 1 - half)

            def hash_block(pb):
                @pl.loop(pb * BG, (pb + 1) * BG)
                def _group(g):
                    o = g * L
                    xa = x_buf[pl.ds(xoff + o, L)]
                    xb = x_buf[pl.ds(xoff + CP + o, L)]
                    xc = x_buf[pl.ds(xoff + 2 * CP + o, L)]
                    for lvl in range(NUM_LEVELS):
                        res = float(RESOLUTIONS[lvl])
                        ia = (xa * res).astype(jnp.int32)
                        ib = (xb * res).astype(jnp.int32)
                        ic = (xc * res).astype(jnp.int32)
                        h = (ia ^ (ib * P1) ^ (ic * P2)) & (HASHMAP_SIZE - 1)
                        idx_buf[pl.ds(lvl * CP + o, L)] = h | (lvl << 15)

            def fire_block(pb, sem):
                return [pltpu.async_copy(
                    tables.at[idx_buf.at[pl.ds(lvl * CP + pb * BP, BP)]],
                    word_buf.at[pl.ds(lvl * CP + pb * BP, BP)], sem)
                    for lvl in range(NUM_LEVELS)]

            def unpack_block(pb):
                @pl.loop(pb * BG, (pb + 1) * BG, unroll=2)
                def _unpack(g):
                    o = g * L
                    for lvl in range(NUM_LEVELS):
                        w = word_buf[pl.ds(lvl * CP + o, L)]
                        out_buf[2 * lvl, pl.ds(o, L)] = (
                            plsc.bitcast(w << 16, jnp.float32))
                        out_buf[2 * lvl + 1, pl.ds(o, L)] = (
                            plsc.bitcast(w & jnp.int32(-65536), jnp.float32))

            def wait_gathers(sem):
                pltpu.make_async_copy(
                    emb_hbm.at[pl.ds(0, NUM_LEVELS * BP)],
                    word_buf.at[pl.ds(0, NUM_LEVELS * BP)], sem).wait()

            def wait_one_out():
                pltpu.make_async_copy(
                    out_buf.at[:, pl.ds(0, BP)],
                    out_hbm.at[:, pl.ds(0, BP)], osem).wait()

            def fire_out(pb, pbase):
                pltpu.async_copy(
                    out_buf.at[:, pl.ds(pb * BP, BP)],
                    out_hbm.at[:, pl.ds(pbase + pb * BP, BP)], osem)

            sems = (gsem0, gsem1)
            for pb in range(PB):
                hash_block(pb)
                fire_block(pb, sems[pb % 2])
                if pb > 0:
                    wait_gathers(sems[(pb - 1) % 2])

                    @pl.when(ci > 0)
                    def _drain_o():
                        wait_one_out()
                    unpack_block(pb - 1)
                    fire_out(pb - 1, p0)
                else:
                    @pl.when(ci > 0)
                    def _prev_tail():
                        wait_gathers(sems[(PB - 1) % 2])
                        wait_one_out()
                        unpack_block(PB - 1)
                        fire_out(PB - 1, p0 - CP)

        pltpu.make_async_copy(
            emb_hbm.at[pl.ds(0, NUM_LEVELS * BP)],
            word_buf.at[pl.ds(0, NUM_LEVELS * BP)],
            gsem1 if (PB - 1) % 2 else gsem0).wait()
        unpack_block_last = PB - 1
        o_last = base + (NCHUNK - 1) * CP

        @pl.loop(unpack_block_last * BG, (unpack_block_last + 1) * BG,
                 unroll=2)
        def _unpack_tail(g):
            o = g * L
            for lvl in range(NUM_LEVELS):
                w = word_buf[pl.ds(lvl * CP + o, L)]
                out_buf[2 * lvl, pl.ds(o, L)] = (
                    plsc.bitcast(w << 16, jnp.float32))
                out_buf[2 * lvl + 1, pl.ds(o, L)] = (
                    plsc.bitcast(w & jnp.int32(-65536), jnp.float32))

        pltpu.async_copy(
            out_buf.at[:, pl.ds((PB - 1) * BP, BP)],
            out_hbm.at[:, pl.ds(o_last + (PB - 1) * BP, BP)], osem)
        pltpu.make_async_copy(
            out_buf, out_hbm.at[:, pl.ds(0, CP)], osem).wait()

    return run(xt_flat, emb_words)


def kernel(x, embeddings):
    xt_flat = x.T.reshape(3 * N)
    emb_words = lax.bitcast_convert_type(
        embeddings.astype(jnp.bfloat16).reshape(TBL, 2), jnp.int32)
    out = _embed(xt_flat, emb_words)
    return out.T


# speedup vs baseline: 1.2376x; 1.0703x over previous
import functools

import jax
import jax.numpy as jnp
import numpy as np
from jax import lax
from jax.experimental import pallas as pl
from jax.experimental.pallas import tpu as pltpu
from jax.experimental.pallas import tpu_sc as plsc

NUM_LEVELS = 12
HASHMAP_SIZE = 2 ** 15
_b = np.exp((np.log(512) - np.log(16)) / (NUM_LEVELS - 1))
RESOLUTIONS = [int(16 * _b ** i) for i in range(NUM_LEVELS)]
P1 = 2654435761 & 0x7FFF
P2 = 805459861 & 0x7FFF

N = 524288
NC, NS, L = 2, 16, 16
NW = NC * NS
PTS_PER_W = N // NW
CP = 1024
NCHUNK = PTS_PER_W // CP
GROUPS = CP // L
ROWS = CP * NUM_LEVELS
PB = 4
BP = CP // PB
BG = GROUPS // PB
TBL = NUM_LEVELS * HASHMAP_SIZE
TSLC = TBL // NS


def _embed(xt_flat, emb_words):
    mesh = plsc.VectorSubcoreMesh(
        core_axis_name="c", subcore_axis_name="s",
        num_cores=NC, num_subcores=NS)

    @functools.partial(
        pl.kernel,
        out_type=jax.ShapeDtypeStruct((NUM_LEVELS * 2, N), jnp.float32),
        mesh=mesh,
        compiler_params=pltpu.CompilerParams(
            needs_layout_passes=False, use_tc_tiling_on_sc=False),
        scratch_types=[
            pltpu.VMEM_SHARED((TBL,), jnp.int32),
            pltpu.VMEM((2 * 3 * CP,), jnp.float32),
            pltpu.VMEM((ROWS,), jnp.int32),
            pltpu.VMEM((ROWS,), jnp.int32),
            pltpu.VMEM((NUM_LEVELS * 2, CP), jnp.float32),
            pltpu.SemaphoreType.DMA,
            pltpu.SemaphoreType.DMA,
            pltpu.SemaphoreType.DMA,
            pltpu.SemaphoreType.DMA,
        ],
    )
    def run(x_hbm, emb_hbm, out_hbm,
            tables, x_buf, idx_buf, word_buf, out_buf,
            gsem0, gsem1, xsem, osem):
        cid = lax.axis_index("c")
        sid = lax.axis_index("s")
        wid = sid * NC + cid

        base = wid * PTS_PER_W

        def fire_x(ci, half):
            p0 = base + ci * CP
            for c in range(3):
                pltpu.async_copy(x_hbm.at[pl.ds(c * N + p0, CP)],
                                 x_buf.at[pl.ds(half * (3 * CP) + c * CP,
                                                CP)], xsem)

        fire_x(0, 0)
        pltpu.sync_copy(emb_hbm.at[pl.ds(sid * TSLC, TSLC)],
                        tables.at[pl.ds(sid * TSLC, TSLC)])
        plsc.subcore_barrier()

        @pl.loop(0, NCHUNK)
        def _chunk(ci):
            p0 = base + ci * CP
            half = ci & 1
            xoff = half * (3 * CP)
            pltpu.make_async_copy(
                x_hbm.at[pl.ds(0, 3 * CP)],
                x_buf.at[pl.ds(xoff, 3 * CP)], xsem).wait()

            @pl.when(ci < NCHUNK - 1)
            def _prefetch():
                fire_x(ci + 1, 1 - half)

            def hash_block(pb):
                @pl.loop(pb * BG, (pb + 1) * BG, unroll=2)
                def _group(g):
                    o = g * L
                    xa = x_buf[pl.ds(xoff + o, L)]
                    xb = x_buf[pl.ds(xoff + CP + o, L)]
                    xc = x_buf[pl.ds(xoff + 2 * CP + o, L)]
                    for lvl in range(NUM_LEVELS):
                        res = float(RESOLUTIONS[lvl])
                        ia = (xa * res).astype(jnp.int32)
                        ib = (xb * res).astype(jnp.int32)
                        ic = (xc * res).astype(jnp.int32)
                        h = (ia ^ (ib * P1) ^ (ic * P2)) & (HASHMAP_SIZE - 1)
                        idx_buf[pl.ds(lvl * CP + o, L)] = h | (lvl << 15)

            def fire_block(pb, sem):
                return [pltpu.async_copy(
                    tables.at[idx_buf.at[pl.ds(lvl * CP + pb * BP, BP)]],
                    word_buf.at[pl.ds(lvl * CP + pb * BP, BP)], sem)
                    for lvl in range(NUM_LEVELS)]

            def unpack_block(pb):
                @pl.loop(pb * BG, (pb + 1) * BG, unroll=2)
                def _unpack(g):
                    o = g * L
                    for lvl in range(NUM_LEVELS):
                        w = word_buf[pl.ds(lvl * CP + o, L)]
                        out_buf[2 * lvl, pl.ds(o, L)] = (
                            plsc.bitcast(w << 16, jnp.float32))
                        out_buf[2 * lvl + 1, pl.ds(o, L)] = (
                            plsc.bitcast(w & jnp.int32(-65536), jnp.float32))

            def wait_gathers(sem):
                pltpu.make_async_copy(
                    emb_hbm.at[pl.ds(0, NUM_LEVELS * BP)],
                    word_buf.at[pl.ds(0, NUM_LEVELS * BP)], sem).wait()

            def wait_one_out():
                pltpu.make_async_copy(
                    out_buf.at[:, pl.ds(0, BP)],
                    out_hbm.at[:, pl.ds(0, BP)], osem).wait()

            def fire_out(pb, pbase):
                pltpu.async_copy(
                    out_buf.at[:, pl.ds(pb * BP, BP)],
                    out_hbm.at[:, pl.ds(pbase + pb * BP, BP)], osem)

            sems = (gsem0, gsem1)
            for pb in range(PB):
                hash_block(pb)
                fire_block(pb, sems[pb % 2])
                if pb > 0:
                    wait_gathers(sems[(pb - 1) % 2])

                    @pl.when(ci > 0)
                    def _drain_o():
                        wait_one_out()
                    unpack_block(pb - 1)
                    fire_out(pb - 1, p0)
                else:
                    @pl.when(ci > 0)
                    def _prev_tail():
                        wait_gathers(sems[(PB - 1) % 2])
                        wait_one_out()
                        unpack_block(PB - 1)
                        fire_out(PB - 1, p0 - CP)

        pltpu.make_async_copy(
            emb_hbm.at[pl.ds(0, NUM_LEVELS * BP)],
            word_buf.at[pl.ds(0, NUM_LEVELS * BP)],
            gsem1 if (PB - 1) % 2 else gsem0).wait()
        unpack_block_last = PB - 1
        o_last = base + (NCHUNK - 1) * CP

        @pl.loop(unpack_block_last * BG, (unpack_block_last + 1) * BG,
                 unroll=2)
        def _unpack_tail(g):
            o = g * L
            for lvl in range(NUM_LEVELS):
                w = word_buf[pl.ds(lvl * CP + o, L)]
                out_buf[2 * lvl, pl.ds(o, L)] = (
                    plsc.bitcast(w << 16, jnp.float32))
                out_buf[2 * lvl + 1, pl.ds(o, L)] = (
                    plsc.bitcast(w & jnp.int32(-65536), jnp.float32))

        pltpu.async_copy(
            out_buf.at[:, pl.ds((PB - 1) * BP, BP)],
            out_hbm.at[:, pl.ds(o_last + (PB - 1) * BP, BP)], osem)
        pltpu.make_async_copy(
            out_buf, out_hbm.at[:, pl.ds(0, CP)], osem).wait()

    return run(xt_flat, emb_words)


def kernel(x, embeddings):
    xt_flat = x.T.reshape(3 * N)
    emb_words = lax.bitcast_convert_type(
        embeddings.astype(jnp.bfloat16).reshape(TBL, 2), jnp.int32)
    out = _embed(xt_flat, emb_words)
    return out.T
